# Initial kernel scaffold; baseline (speedup 1.0000x reference)
import jax
import jax.numpy as jnp
from jax.experimental import pallas as pl


def kernel(node_pos, full_edge_index, full_edge_attr, batch_vec, edge_index):
    raise NotImplementedError("write your pallas kernel here")


# speedup vs baseline: 210.2561x; 210.2561x over previous
import functools

import jax
import jax.numpy as jnp
from jax import lax
from jax.experimental import pallas as pl
from jax.experimental.pallas import tpu as pltpu
from jax.experimental.pallas import tpu_sc as plsc

_N_NODES = 50000
_E = 1600000
_N_GRAPHS = 16
_NW = 32
_PER_W = _E // _NW
_B = 2000
_NCHUNK = _PER_W // _B
_G = _B // 16
_U = 5

_MESH = plsc.VectorSubcoreMesh(core_axis_name="c", subcore_axis_name="s")


@functools.partial(
    pl.kernel,
    out_type=jax.ShapeDtypeStruct((_NW, 16), jnp.float32),
    mesh=_MESH,
    scratch_types=[
        pltpu.VMEM((2 * _N_NODES,), jnp.float32),
        pltpu.VMEM((_B,), jnp.int32),
        pltpu.VMEM((_B,), jnp.int32),
        pltpu.VMEM((_B,), jnp.float32),
        pltpu.VMEM((_B,), jnp.int32),
        pltpu.VMEM((_B,), jnp.int32),
        pltpu.VMEM((_B,), jnp.float32),
        pltpu.VMEM((16,), jnp.float32),
        pltpu.SemaphoreType.DMA,
        pltpu.SemaphoreType.DMA,
        pltpu.SemaphoreType.DMA,
    ],
    compiler_params=pltpu.CompilerParams(needs_layout_passes=False),
)
def _stress_sc(nodes_hbm, fei_hbm, d_hbm, out_hbm, np_v,
               src0_v, dst0_v, d0_v, src1_v, dst1_v, d1_v,
               acc_v, nsem, sem0, sem1):
    wid = lax.axis_index("s") * 2 + lax.axis_index("c")
    base = wid * _PER_W
    bufs = ((src0_v, dst0_v, d0_v, sem0), (src1_v, dst1_v, d1_v, sem1))

    def issue(c, buf):
        sv, dv, ddv, sem = buf
        off = base + c * _B
        pltpu.make_async_copy(fei_hbm.at[pl.ds(off, _B)], sv, sem).start()
        pltpu.make_async_copy(fei_hbm.at[pl.ds(_E + off, _B)], dv, sem).start()
        pltpu.make_async_copy(d_hbm.at[pl.ds(off, _B)], ddv, sem).start()

    def drain(buf):
        sv, dv, ddv, sem = buf
        pltpu.make_async_copy(fei_hbm.at[pl.ds(0, _B)], sv, sem).wait()
        pltpu.make_async_copy(fei_hbm.at[pl.ds(0, _B)], dv, sem).wait()
        pltpu.make_async_copy(d_hbm.at[pl.ds(0, _B)], ddv, sem).wait()

    def edge_group(g, buf):
        sv, dv, ddv, _ = buf
        s = sv[pl.ds(g * 16, 16)]
        t = dv[pl.ds(g * 16, 16)]
        dd = ddv[pl.ds(g * 16, 16)]
        s2 = s + s
        t2 = t + t
        sx = plsc.load_gather(np_v, [s2])
        sy = plsc.load_gather(np_v, [s2 + 1])
        ex = plsc.load_gather(np_v, [t2])
        ey = plsc.load_gather(np_v, [t2 + 1])
        dx = sx - ex
        dy = sy - ey
        sq = dx * dx + dy * dy
        sqc = jnp.maximum(sq, jnp.float32(1e-30))
        ii = plsc.bitcast(sqc, jnp.int32)
        y = plsc.bitcast(jnp.int32(0x5F3759DF) - (ii >> 1), jnp.float32)
        h = jnp.float32(0.5) * sqc
        y = y * (jnp.float32(1.5) - h * y * y)
        y = y * (jnp.float32(1.5) - h * y * y)
        y = y * (jnp.float32(1.5) - h * y * y)
        eu = sqc * y
        r = (eu - dd) / dd
        return r * r

    def compute(buf, accs):
        def group_body(i, accs):
            return tuple(
                accs[u] + edge_group(i * _U + u, buf) for u in range(_U)
            )
        return lax.fori_loop(0, _G // _U, group_body, accs)

    pltpu.make_async_copy(nodes_hbm, np_v, nsem).start()
    issue(0, bufs[0])
    pltpu.make_async_copy(nodes_hbm, np_v, nsem).wait()

    def pair_body(cc, accs):
        c0 = cc * 2
        drain(bufs[0])
        issue(c0 + 1, bufs[1])
        accs = compute(bufs[0], accs)
        drain(bufs[1])
        issue(c0 + 2, bufs[0])
        return compute(bufs[1], accs)

    zero = jnp.zeros((16,), jnp.float32)
    accs = lax.fori_loop(0, (_NCHUNK - 1) // 2, pair_body, (zero,) * _U)
    drain(bufs[0])
    accs = compute(bufs[0], accs)

    total = accs[0]
    for u in range(1, _U):
        total = total + accs[u]
    acc_v[...] = total * jnp.float32(1.0 / _N_GRAPHS)
    pltpu.sync_copy(acc_v, out_hbm.at[wid])


def kernel(node_pos, full_edge_index, full_edge_attr, batch_vec, edge_index):
    del batch_vec, edge_index
    nodes = node_pos.reshape(-1)
    fei = full_edge_index.astype(jnp.int32).reshape(-1)
    d = full_edge_attr.reshape(-1)
    partials = _stress_sc(nodes, fei, d)
    return jnp.sum(partials)


# speedup vs baseline: 212.8741x; 1.0125x over previous
import functools

import jax
import jax.numpy as jnp
from jax import lax
from jax.experimental import pallas as pl
from jax.experimental.pallas import tpu as pltpu
from jax.experimental.pallas import tpu_sc as plsc

_N_NODES = 50000
_E = 1600000
_N_GRAPHS = 16
_NW = 32
_PER_W = _E // _NW
_B = 2000
_NCHUNK = _PER_W // _B
_G = _B // 16
_U = 5

_MESH = plsc.VectorSubcoreMesh(core_axis_name="c", subcore_axis_name="s")


@functools.partial(
    pl.kernel,
    out_type=jax.ShapeDtypeStruct((_NW, 16), jnp.float32),
    mesh=_MESH,
    scratch_types=[
        pltpu.VMEM((2 * _N_NODES,), jnp.float32),
        pltpu.VMEM((_B,), jnp.int32),
        pltpu.VMEM((_B,), jnp.int32),
        pltpu.VMEM((_B,), jnp.float32),
        pltpu.VMEM((_B,), jnp.int32),
        pltpu.VMEM((_B,), jnp.int32),
        pltpu.VMEM((_B,), jnp.float32),
        pltpu.VMEM((16,), jnp.float32),
        pltpu.SemaphoreType.DMA,
        pltpu.SemaphoreType.DMA,
        pltpu.SemaphoreType.DMA,
    ],
    compiler_params=pltpu.CompilerParams(needs_layout_passes=False),
)
def _stress_sc(nodes_hbm, fei_hbm, d_hbm, out_hbm, np_v,
               src0_v, dst0_v, d0_v, src1_v, dst1_v, d1_v,
               acc_v, nsem, sem0, sem1):
    wid = lax.axis_index("s") * 2 + lax.axis_index("c")
    base = wid * _PER_W
    bufs = ((src0_v, dst0_v, d0_v, sem0), (src1_v, dst1_v, d1_v, sem1))

    def issue(c, buf):
        sv, dv, ddv, sem = buf
        off = base + c * _B
        pltpu.make_async_copy(fei_hbm.at[pl.ds(off, _B)], sv, sem).start()
        pltpu.make_async_copy(fei_hbm.at[pl.ds(_E + off, _B)], dv, sem).start()
        pltpu.make_async_copy(d_hbm.at[pl.ds(off, _B)], ddv, sem).start()

    def drain(buf):
        sv, dv, ddv, sem = buf
        pltpu.make_async_copy(fei_hbm.at[pl.ds(0, _B)], sv, sem).wait()
        pltpu.make_async_copy(fei_hbm.at[pl.ds(0, _B)], dv, sem).wait()
        pltpu.make_async_copy(d_hbm.at[pl.ds(0, _B)], ddv, sem).wait()

    def edge_group(g, buf):
        sv, dv, ddv, _ = buf
        s = sv[pl.ds(g * 16, 16)]
        t = dv[pl.ds(g * 16, 16)]
        dd = ddv[pl.ds(g * 16, 16)]
        s2 = s + s
        t2 = t + t
        sx = plsc.load_gather(np_v, [s2])
        sy = plsc.load_gather(np_v, [s2 + 1])
        ex = plsc.load_gather(np_v, [t2])
        ey = plsc.load_gather(np_v, [t2 + 1])
        dx = sx - ex
        dy = sy - ey
        sq = dx * dx + dy * dy
        sqc = jnp.maximum(sq, jnp.float32(1e-30))
        ii = plsc.bitcast(sqc, jnp.int32)
        y = plsc.bitcast(jnp.int32(0x5F3759DF) - (ii >> 1), jnp.float32)
        h = jnp.float32(0.5) * sqc
        y = y * (jnp.float32(1.5) - h * y * y)
        y = y * (jnp.float32(1.5) - h * y * y)
        eu = sqc * y
        r = (eu - dd) / dd
        return r * r

    def compute(buf, accs):
        def group_body(i, accs):
            return tuple(
                accs[u] + edge_group(i * _U + u, buf) for u in range(_U)
            )
        return lax.fori_loop(0, _G // _U, group_body, accs)

    pltpu.make_async_copy(nodes_hbm, np_v, nsem).start()
    issue(0, bufs[0])
    pltpu.make_async_copy(nodes_hbm, np_v, nsem).wait()

    def pair_body(cc, accs):
        c0 = cc * 2
        drain(bufs[0])
        issue(c0 + 1, bufs[1])
        accs = compute(bufs[0], accs)
        drain(bufs[1])
        issue(c0 + 2, bufs[0])
        return compute(bufs[1], accs)

    zero = jnp.zeros((16,), jnp.float32)
    accs = lax.fori_loop(0, (_NCHUNK - 1) // 2, pair_body, (zero,) * _U)
    drain(bufs[0])
    accs = compute(bufs[0], accs)

    total = accs[0]
    for u in range(1, _U):
        total = total + accs[u]
    acc_v[...] = total * jnp.float32(1.0 / _N_GRAPHS)
    pltpu.sync_copy(acc_v, out_hbm.at[wid])


def kernel(node_pos, full_edge_index, full_edge_attr, batch_vec, edge_index):
    del batch_vec, edge_index
    nodes = node_pos.reshape(-1)
    fei = full_edge_index.astype(jnp.int32).reshape(-1)
    d = full_edge_attr.reshape(-1)
    partials = _stress_sc(nodes, fei, d)
    return jnp.sum(partials)
